# baseline (speedup 1.0000x reference)
import functools

import jax
import jax.numpy as jnp
from jax import lax
from jax.experimental import pallas as pl
from jax.experimental.pallas import tpu as pltpu
from jax.experimental.pallas import tpu_sc as plsc

N = 10000
E = 320000
D = 128
DH = 64
N_P = 10240
JUNK = 10016
E_P = 327680
R0 = 160
R1 = 0
CH = 40
ROWS_DEG = 80
RPT = N_P // 16
RB = 1024
DW = 128

_mesh = plsc.VectorSubcoreMesh(core_axis_name="c", subcore_axis_name="s")


@functools.partial(
    pl.kernel,
    mesh=_mesh,
    out_type=jax.ShapeDtypeStruct((2, N_P, DW), jnp.float32),
    scratch_types=[
        pltpu.VMEM((ROWS_DEG, 128), jnp.int32),
        pltpu.VMEM((128, DW), jnp.float32),
        pltpu.VMEM((128, DW), jnp.float32),
        pltpu.VMEM_SHARED((N_P, DW), jnp.float32),
    ],
)
def _deg_kernel(dst_hbm, ones_hbm, z_hbm, out_hbm, idxbuf, ones_buf, zbuf, deg_sh):
    c = lax.axis_index("c")
    s = lax.axis_index("s")
    wid = c * 16 + s
    pltpu.sync_copy(dst_hbm.at[wid], idxbuf)
    pltpu.sync_copy(ones_hbm, ones_buf)
    pltpu.sync_copy(z_hbm, zbuf)
    for q in range(5):
        pltpu.sync_copy(zbuf, deg_sh.at[pl.ds(s * RPT + q * 128, 128)])
    plsc.subcore_barrier()

    def body(j, carry):
        pltpu.sync_copy(ones_buf, deg_sh.at[idxbuf.at[j]], add=True)
        return carry

    lax.fori_loop(0, ROWS_DEG, body, 0)
    plsc.subcore_barrier()
    pltpu.sync_copy(deg_sh.at[pl.ds(s * RPT, RPT)],
                    out_hbm.at[c, pl.ds(s * RPT, RPT)])


@functools.partial(
    pl.kernel,
    mesh=_mesh,
    out_type=jax.ShapeDtypeStruct((2, N_P, D), jnp.float32),
    scratch_types=[
        pltpu.VMEM((CH, 128), jnp.int32),
        pltpu.VMEM((CH, 128), jnp.int32),
        pltpu.VMEM((128, D), jnp.float32),
        pltpu.VMEM((128, D), jnp.float32),
        pltpu.VMEM_SHARED((N_P, D), jnp.float32),
        pltpu.SemaphoreType.DMA,
        pltpu.SemaphoreType.DMA,
    ],
)
def _agg_kernel(g_hbm, src0_hbm, dst0_hbm, src1_hbm, dst1_hbm, z_hbm, out_hbm,
                srcbuf, dstbuf, buf0, buf1, acc, sem0, sem1):
    c = lax.axis_index("c")
    s = lax.axis_index("s")
    pltpu.sync_copy(z_hbm, buf0)
    for q in range(5):
        pltpu.sync_copy(buf0, acc.at[pl.ds(s * RPT + q * 128, 128)])
    plsc.subcore_barrier()

    bufs = (buf0, buf1)
    sems = (sem0, sem1)

    def run(src_hbm, dst_hbm, rows):
        for hh in range(rows // CH):
            pltpu.sync_copy(src_hbm.at[s, pl.ds(hh * CH, CH)], srcbuf)
            pltpu.sync_copy(dst_hbm.at[s, pl.ds(hh * CH, CH)], dstbuf)
            pltpu.async_copy(g_hbm.at[srcbuf.at[0]], buf0, sem0)

            def body(i, carry):
                j0 = i * 2
                for b in range(2):
                    j = j0 + b

                    @pl.when(j + 1 < CH)
                    def _start():
                        pltpu.async_copy(g_hbm.at[srcbuf.at[j + 1]],
                                         bufs[1 - b], sems[1 - b])

                    pltpu.make_async_copy(g_hbm.at[srcbuf.at[j]],
                                          bufs[b], sems[b]).wait()
                    pltpu.sync_copy(bufs[b], acc.at[dstbuf.at[j]], add=True)
                return carry

            lax.fori_loop(0, CH // 2, body, 0)

    @pl.when(c == 0)
    def _run0():
        run(src0_hbm, dst0_hbm, R0)

    @pl.when(c == 1)
    def _run1():
        run(src1_hbm, dst1_hbm, R1)

    plsc.subcore_barrier()
    pltpu.sync_copy(acc.at[pl.ds(s * RPT, RPT)],
                    out_hbm.at[c, pl.ds(s * RPT, RPT)])


def _tc_a_body(x_ref, w_ref, d0_ref, d1_ref, ga_ref, dinv_ref):
    h = jnp.dot(x_ref[...], w_ref[...], preferred_element_type=jnp.float32)
    deg = d0_ref[...][:, 0:1] + d1_ref[...][:, 0:1] + 1.0
    dinv = lax.rsqrt(deg)
    dinv_ref[...] = dinv
    ga_ref[...] = h * dinv


def _tc_b_body(s0_ref, s1_ref, g_ref, di_ref, b_ref, w_ref, og_ref):
    sv = s0_ref[...] + s1_ref[...]
    di = di_ref[...]
    h = jnp.maximum(di * (sv + g_ref[...]) + b_ref[...], 0.0)
    h2 = jnp.dot(h, w_ref[...], preferred_element_type=jnp.float32)
    og_ref[...] = h2 * di


def _tc_c_body(x_ref, s0_ref, s1_ref, g_ref, di_ref, b_ref, o_ref):
    conv = di_ref[...] * (s0_ref[...] + s1_ref[...] + g_ref[...]) + b_ref[...]
    o_ref[...] = jnp.maximum(x_ref[...] + conv, 0.0)


def _row_spec(w):
    return pl.BlockSpec((RB, w), lambda i: (i, 0))


def _fix_spec(shape):
    return pl.BlockSpec(shape, lambda i: (0, 0))


def kernel(node_feature, edge_index, W1, b1, W2, b2):
    f32 = jnp.float32
    src = edge_index[0]
    dst = edge_index[1]
    pad_e = E_P - E
    srcp = jnp.concatenate([src, jnp.zeros((pad_e,), jnp.int32)])
    dstp = jnp.concatenate([dst, jnp.full((pad_e,), JUNK, jnp.int32)])
    src2d = srcp.reshape(-1, 128)
    dst2d = dstp.reshape(-1, 128)
    n0 = 16 * R0
    src_a0 = src2d[:n0].reshape(16, R0, 128)
    dst_a0 = dst2d[:n0].reshape(16, R0, 128)
    src_a1 = src_a0
    dst_a1 = dst_a0
    dst_deg = dstp.reshape(32, ROWS_DEG, 128)
    x_p = jnp.concatenate(
        [node_feature, jnp.zeros((N_P - N, D), f32)], axis=0)
    ones16 = jnp.ones((128, DW), f32)
    zeros16 = jnp.zeros((128, DW), f32)
    zerosD = jnp.zeros((128, D), f32)
    b1r = b1.reshape(1, D)
    b2r = b2.reshape(1, D)

    deg_out = _deg_kernel(dst_deg, ones16, zeros16)
    d0 = deg_out[0]
    d1 = deg_out[1]

    grid = N_P // RB
    g1, dinv = pl.pallas_call(
        _tc_a_body,
        grid=(grid,),
        in_specs=[_row_spec(D), _fix_spec((D, D)), _row_spec(DW), _row_spec(DW)],
        out_specs=[_row_spec(D), _row_spec(1)],
        out_shape=[jax.ShapeDtypeStruct((N_P, D), f32),
                   jax.ShapeDtypeStruct((N_P, 1), f32)],
    )(x_p, W1, d0, d1)

    s1 = _agg_kernel(g1, src_a0, dst_a0, src_a1, dst_a1, zerosD)

    g2 = pl.pallas_call(
        _tc_b_body,
        grid=(grid,),
        in_specs=[_row_spec(D), _row_spec(D), _row_spec(D),
                  _row_spec(1), _fix_spec((1, D)), _fix_spec((D, D))],
        out_specs=_row_spec(D),
        out_shape=jax.ShapeDtypeStruct((N_P, D), f32),
    )(s1[0], s1[1], g1, dinv, b1r, W2)

    s2 = _agg_kernel(g2, src_a0, dst_a0, src_a1, dst_a1, zerosD)

    out = pl.pallas_call(
        _tc_c_body,
        grid=(grid,),
        in_specs=[_row_spec(D), _row_spec(D), _row_spec(D), _row_spec(D),
                  _row_spec(1), _fix_spec((1, D))],
        out_specs=_row_spec(D),
        out_shape=jax.ShapeDtypeStruct((N, D), f32),
    )(x_p, s2[0], s2[1], g2, dinv, b2r)

    return out


# speedup vs baseline: 3.3693x; 3.3693x over previous
import functools

import jax
import jax.numpy as jnp
from jax import lax
from jax.experimental import pallas as pl
from jax.experimental.pallas import tpu as pltpu
from jax.experimental.pallas import tpu_sc as plsc

N = 10000
E = 320000
D = 128
DH = 64
N_P = 10240
JUNK = 10016
E_P = 327680
R0 = 80
R1 = 80
CH = 40
ROWS_DEG = 80
RPT = N_P // 16
RB = 1024
DW = 128

_mesh = plsc.VectorSubcoreMesh(core_axis_name="c", subcore_axis_name="s")


@functools.partial(
    pl.kernel,
    mesh=_mesh,
    out_type=jax.ShapeDtypeStruct((2, N_P, DW), jnp.float32),
    scratch_types=[
        pltpu.VMEM((ROWS_DEG, 128), jnp.int32),
        pltpu.VMEM((128, DW), jnp.float32),
        pltpu.VMEM((128, DW), jnp.float32),
        pltpu.VMEM_SHARED((N_P, DW), jnp.float32),
    ],
)
def _deg_kernel(dst_hbm, ones_hbm, z_hbm, out_hbm, idxbuf, ones_buf, zbuf, deg_sh):
    c = lax.axis_index("c")
    s = lax.axis_index("s")
    wid = c * 16 + s
    pltpu.sync_copy(dst_hbm.at[wid], idxbuf)
    pltpu.sync_copy(ones_hbm, ones_buf)
    pltpu.sync_copy(z_hbm, zbuf)
    for q in range(5):
        pltpu.sync_copy(zbuf, deg_sh.at[pl.ds(s * RPT + q * 128, 128)])
    plsc.subcore_barrier()

    def body(j, carry):
        pltpu.sync_copy(ones_buf, deg_sh.at[idxbuf.at[j]], add=True)
        return carry

    lax.fori_loop(0, ROWS_DEG, body, 0)
    plsc.subcore_barrier()
    pltpu.sync_copy(deg_sh.at[pl.ds(s * RPT, RPT)],
                    out_hbm.at[c, pl.ds(s * RPT, RPT)])


@functools.partial(
    pl.kernel,
    mesh=_mesh,
    out_type=jax.ShapeDtypeStruct((2, N_P, D), jnp.float32),
    scratch_types=[
        pltpu.VMEM((CH, 128), jnp.int32),
        pltpu.VMEM((CH, 128), jnp.int32),
        pltpu.VMEM((128, D), jnp.float32),
        pltpu.VMEM((128, D), jnp.float32),
        pltpu.VMEM_SHARED((N_P, D), jnp.float32),
        pltpu.SemaphoreType.DMA,
        pltpu.SemaphoreType.DMA,
    ],
)
def _agg_kernel(g_hbm, src0_hbm, dst0_hbm, src1_hbm, dst1_hbm, z_hbm, out_hbm,
                srcbuf, dstbuf, buf0, buf1, acc, sem0, sem1):
    c = lax.axis_index("c")
    s = lax.axis_index("s")
    pltpu.sync_copy(z_hbm, buf0)
    for q in range(5):
        pltpu.sync_copy(buf0, acc.at[pl.ds(s * RPT + q * 128, 128)])
    plsc.subcore_barrier()

    bufs = (buf0, buf1)
    sems = (sem0, sem1)

    def run(src_hbm, dst_hbm, rows):
        for hh in range(rows // CH):
            pltpu.sync_copy(src_hbm.at[s, pl.ds(hh * CH, CH)], srcbuf)
            pltpu.sync_copy(dst_hbm.at[s, pl.ds(hh * CH, CH)], dstbuf)
            pltpu.async_copy(g_hbm.at[srcbuf.at[0]], buf0, sem0)

            def body(i, carry):
                j0 = i * 2
                for b in range(2):
                    j = j0 + b

                    @pl.when(j + 1 < CH)
                    def _start():
                        pltpu.async_copy(g_hbm.at[srcbuf.at[j + 1]],
                                         bufs[1 - b], sems[1 - b])

                    pltpu.make_async_copy(g_hbm.at[srcbuf.at[j]],
                                          bufs[b], sems[b]).wait()
                    pltpu.sync_copy(bufs[b], acc.at[dstbuf.at[j]], add=True)
                return carry

            lax.fori_loop(0, CH // 2, body, 0)

    @pl.when(c == 0)
    def _run0():
        run(src0_hbm, dst0_hbm, R0)

    @pl.when(c == 1)
    def _run1():
        run(src1_hbm, dst1_hbm, R1)

    plsc.subcore_barrier()
    pltpu.sync_copy(acc.at[pl.ds(s * RPT, RPT)],
                    out_hbm.at[c, pl.ds(s * RPT, RPT)])


def _tc_a_body(x_ref, w_ref, d0_ref, d1_ref, ga_ref, dinv_ref):
    h = jnp.dot(x_ref[...], w_ref[...], preferred_element_type=jnp.float32)
    deg = d0_ref[...][:, 0:1] + d1_ref[...][:, 0:1] + 1.0
    dinv = lax.rsqrt(deg)
    dinv_ref[...] = dinv
    ga_ref[...] = h * dinv


def _tc_b_body(s0_ref, s1_ref, g_ref, di_ref, b_ref, w_ref, og_ref):
    sv = s0_ref[...] + s1_ref[...]
    di = di_ref[...]
    h = jnp.maximum(di * (sv + g_ref[...]) + b_ref[...], 0.0)
    h2 = jnp.dot(h, w_ref[...], preferred_element_type=jnp.float32)
    og_ref[...] = h2 * di


def _tc_c_body(x_ref, s0_ref, s1_ref, g_ref, di_ref, b_ref, o_ref):
    conv = di_ref[...] * (s0_ref[...] + s1_ref[...] + g_ref[...]) + b_ref[...]
    o_ref[...] = jnp.maximum(x_ref[...] + conv, 0.0)


def _row_spec(w):
    return pl.BlockSpec((RB, w), lambda i: (i, 0))


def _fix_spec(shape):
    return pl.BlockSpec(shape, lambda i: (0, 0))


def kernel(node_feature, edge_index, W1, b1, W2, b2):
    f32 = jnp.float32
    src = edge_index[0]
    dst = edge_index[1]
    pad_e = E_P - E
    pad_i = jnp.arange(pad_e, dtype=jnp.int32)
    srcp = jnp.concatenate([src, pad_i % N])
    dstp = jnp.concatenate([dst, N + pad_i % (N_P - N)])
    src2d = srcp.reshape(-1, 128)
    dst2d = dstp.reshape(-1, 128)
    n0 = 16 * R0
    src_a0 = src2d[:n0].reshape(16, R0, 128)
    dst_a0 = dst2d[:n0].reshape(16, R0, 128)
    src_a1 = src2d[n0:].reshape(16, R1, 128)
    dst_a1 = dst2d[n0:].reshape(16, R1, 128)
    dst_deg = dstp.reshape(32, ROWS_DEG, 128)
    x_p = jnp.concatenate(
        [node_feature, jnp.zeros((N_P - N, D), f32)], axis=0)
    ones16 = jnp.ones((128, DW), f32)
    zeros16 = jnp.zeros((128, DW), f32)
    zerosD = jnp.zeros((128, D), f32)
    b1r = b1.reshape(1, D)
    b2r = b2.reshape(1, D)

    deg_out = _deg_kernel(dst_deg, ones16, zeros16)
    d0 = deg_out[0]
    d1 = deg_out[1]

    grid = N_P // RB
    g1, dinv = pl.pallas_call(
        _tc_a_body,
        grid=(grid,),
        in_specs=[_row_spec(D), _fix_spec((D, D)), _row_spec(DW), _row_spec(DW)],
        out_specs=[_row_spec(D), _row_spec(1)],
        out_shape=[jax.ShapeDtypeStruct((N_P, D), f32),
                   jax.ShapeDtypeStruct((N_P, 1), f32)],
    )(x_p, W1, d0, d1)

    s1 = _agg_kernel(g1, src_a0, dst_a0, src_a1, dst_a1, zerosD)

    g2 = pl.pallas_call(
        _tc_b_body,
        grid=(grid,),
        in_specs=[_row_spec(D), _row_spec(D), _row_spec(D),
                  _row_spec(1), _fix_spec((1, D)), _fix_spec((D, D))],
        out_specs=_row_spec(D),
        out_shape=jax.ShapeDtypeStruct((N_P, D), f32),
    )(s1[0], s1[1], g1, dinv, b1r, W2)

    s2 = _agg_kernel(g2, src_a0, dst_a0, src_a1, dst_a1, zerosD)

    out = pl.pallas_call(
        _tc_c_body,
        grid=(grid,),
        in_specs=[_row_spec(D), _row_spec(D), _row_spec(D), _row_spec(D),
                  _row_spec(1), _fix_spec((1, D))],
        out_specs=_row_spec(D),
        out_shape=jax.ShapeDtypeStruct((N, D), f32),
    )(x_p, s2[0], s2[1], g2, dinv, b2r)

    return out
